# baseline (speedup 1.0000x reference)
import functools

import jax
import jax.numpy as jnp
from jax import lax
from jax.experimental import pallas as pl
from jax.experimental.pallas import tpu as pltpu
from jax.experimental.pallas import tpu_sc as plsc

V = 1000
LSE_PAD = 1024
NUM_CORES = 2
NUM_SUBCORES = 16
LANES = 16
NW = NUM_CORES * NUM_SUBCORES


def _lse_body(table_ref, out_ref):
    x = table_ref[...]
    m = jnp.max(x, axis=1, keepdims=True)
    s = jnp.sum(jnp.exp(x - m), axis=1, keepdims=True)
    lse = m + jnp.log(s)
    pad = jnp.zeros((LSE_PAD - V, 1), jnp.float32)
    out_ref[...] = jnp.concatenate([lse, pad], axis=0)


def _compute_lse(table):
    out = pl.pallas_call(
        _lse_body,
        out_shape=jax.ShapeDtypeStruct((LSE_PAD, 1), jnp.float32),
    )(table)
    return out.reshape(LSE_PAD)


def _loss_body(part_ref, out_ref, *, n_tokens):
    val = jnp.sum(part_ref[...]) * (1.0 / n_tokens)
    out_ref[...] = jnp.broadcast_to(val, (1, 1))


def _compute_loss(partials, n_tokens):
    out = pl.pallas_call(
        functools.partial(_loss_body, n_tokens=n_tokens),
        out_shape=jax.ShapeDtypeStruct((1, 1), jnp.float32),
    )(partials)
    return out[0, 0]


def _make_sc_gather(n_tokens, chunk):
    assert n_tokens % (8 * NW) == 0
    per_w = n_tokens // NW
    nbuf = 2
    assert per_w % chunk == 0
    n_chunks = per_w // chunk
    assert n_chunks % nbuf == 0 and chunk % 8 == 0
    groups = (chunk + LANES - 1) // LANES

    mesh = plsc.VectorSubcoreMesh(
        core_axis_name="c", subcore_axis_name="s",
        num_cores=NUM_CORES, num_subcores=NUM_SUBCORES)

    @functools.partial(
        pl.kernel,
        mesh=mesh,
        compiler_params=pltpu.CompilerParams(
            use_tc_tiling_on_sc=False, needs_layout_passes=False),
        out_type=[
            jax.ShapeDtypeStruct((n_tokens * V,), jnp.float32),
            jax.ShapeDtypeStruct((NW, LANES), jnp.float32),
        ],
        scratch_types=(
            [pltpu.VMEM((per_w + LANES,), jnp.int32)] * 2
            + [pltpu.VMEM((chunk, V), jnp.float32)] * nbuf
            + [pltpu.VMEM((LANES,), jnp.float32)]
            + [pltpu.VMEM((LSE_PAD,), jnp.float32)]
            + [pltpu.SemaphoreType.DMA] * nbuf
            + [pltpu.SemaphoreType.DMA] * nbuf
        ),
    )
    def sc_gather(table_hbm, idx_hbm, tgt_hbm, lse_hbm, out_hbm, part_hbm,
                  idx_v, tgt_v, *rest):
        rows = rest[:nbuf]
        acc_v, lse_v = rest[nbuf], rest[nbuf + 1]
        gsem = rest[nbuf + 2:nbuf + 2 + nbuf]
        ssem = rest[nbuf + 2 + nbuf:]
        wid = lax.axis_index("s") * NUM_CORES + lax.axis_index("c")
        base_w = pl.multiple_of(wid * per_w, per_w)
        pltpu.sync_copy(lse_hbm, lse_v)
        pltpu.sync_copy(idx_hbm.at[pl.ds(base_w, per_w)],
                        idx_v.at[pl.ds(0, per_w)])
        pltpu.sync_copy(tgt_hbm.at[pl.ds(base_w, per_w)],
                        tgt_v.at[pl.ds(0, per_w)])
        acc_v[...] = jnp.zeros((LANES,), jnp.float32)

        def start_gather(g, b):
            off = pl.multiple_of(g * chunk, chunk)
            pltpu.async_copy(
                table_hbm.at[idx_v.at[pl.ds(off, chunk)]], rows[b], gsem[b])

        def wait_gather(b):
            pltpu.make_async_copy(
                table_hbm.at[pl.ds(0, chunk)], rows[b], gsem[b]).wait()

        def start_scatter(g, b):
            off = pl.multiple_of((base_w + g * chunk) * V, V)
            for i in range(chunk):
                pltpu.async_copy(
                    rows[b].at[i], out_hbm.at[pl.ds(off + i * V, V)], ssem[b])

        def wait_scatter(b):
            for i in range(chunk):
                pltpu.make_async_copy(
                    rows[b].at[i], out_hbm.at[pl.ds(0, V)], ssem[b]).wait()

        def loss_partial(g, b):
            part = jnp.zeros((LANES,), jnp.float32)
            for j in range(groups):
                off = pl.multiple_of(g * chunk + j * LANES, 8)
                rid16 = lax.iota(jnp.int32, LANES) + (j * LANES)
                ok = rid16 < chunk
                rid16 = jnp.where(ok, rid16, 0)
                idx16 = jnp.where(ok, idx_v[pl.ds(off, LANES)], 0)
                tgt16 = jnp.where(ok, tgt_v[pl.ds(off, LANES)], 0)
                lse16 = plsc.load_gather(lse_v, [idx16])
                picked = plsc.load_gather(rows[b], [rid16, tgt16])
                part = part + jnp.where(ok, lse16 - picked, 0.0)
            acc_v[...] = acc_v[...] + part

        for b in range(nbuf):
            start_gather(b, b)

        def body(k, carry):
            for b in range(nbuf):
                g = k * nbuf + b
                wait_gather(b)
                start_scatter(g, b)
                loss_partial(g, b)

                @pl.when(g + nbuf < n_chunks)
                def _():
                    wait_scatter(b)
                    start_gather(g + nbuf, b)
            return carry

        lax.fori_loop(0, n_chunks // nbuf, body, 0)
        for b in range(nbuf):
            wait_scatter(b)
        pltpu.sync_copy(acc_v, part_hbm.at[wid])

    return sc_gather


SPLIT = 4


def kernel(index, targets, table):
    b, t = index.shape
    n_tokens = b * t
    idx_flat = index.reshape(n_tokens).astype(jnp.int32)
    tgt_flat = targets.reshape(n_tokens).astype(jnp.int32)
    lse = _compute_lse(table)
    tok_k = n_tokens // SPLIT
    sc_gather = _make_sc_gather(tok_k, chunk=40)
    pieces, parts = [], []
    for kk in range(SPLIT):
        sl = slice(kk * tok_k, (kk + 1) * tok_k)
        flat_k, part_k = sc_gather(table, idx_flat[sl], tgt_flat[sl], lse)
        pieces.append(flat_k.reshape(b // SPLIT, t, V))
        parts.append(part_k)
    logits = jnp.concatenate(pieces, axis=0)
    loss = _compute_loss(jnp.concatenate(parts, axis=0), n_tokens)
    return logits, loss


# speedup vs baseline: 1.3071x; 1.3071x over previous
import functools

import jax
import jax.numpy as jnp
from jax import lax
from jax.experimental import pallas as pl
from jax.experimental.pallas import tpu as pltpu
from jax.experimental.pallas import tpu_sc as plsc

V = 1000
VP = 1024
LSE_PAD = 1024
NUM_CORES = 2
NUM_SUBCORES = 16
LANES = 16
NW = NUM_CORES * NUM_SUBCORES


def _lse_body(table_ref, out_ref):
    x = table_ref[...]
    m = jnp.max(x, axis=1, keepdims=True)
    s = jnp.sum(jnp.exp(x - m), axis=1, keepdims=True)
    lse = m + jnp.log(s)
    pad = jnp.zeros((LSE_PAD - V, 1), jnp.float32)
    out_ref[...] = jnp.concatenate([lse, pad], axis=0)


def _compute_lse(table):
    out = pl.pallas_call(
        _lse_body,
        out_shape=jax.ShapeDtypeStruct((LSE_PAD, 1), jnp.float32),
    )(table)
    return out.reshape(LSE_PAD)


def _loss_body(part_ref, out_ref, *, n_tokens):
    val = jnp.sum(part_ref[...]) * (1.0 / n_tokens)
    out_ref[...] = jnp.broadcast_to(val, (1, 1))


def _compute_loss(partials, n_tokens):
    out = pl.pallas_call(
        functools.partial(_loss_body, n_tokens=n_tokens),
        out_shape=jax.ShapeDtypeStruct((1, 1), jnp.float32),
    )(partials)
    return out[0, 0]


def _make_sc_gather(n_tokens, chunk):
    assert n_tokens % (8 * NW) == 0
    per_w = n_tokens // NW
    nbuf = 2
    assert per_w % chunk == 0
    n_chunks = per_w // chunk
    assert n_chunks % nbuf == 0 and chunk % 8 == 0
    groups = (chunk + LANES - 1) // LANES

    mesh = plsc.VectorSubcoreMesh(
        core_axis_name="c", subcore_axis_name="s",
        num_cores=NUM_CORES, num_subcores=NUM_SUBCORES)

    @functools.partial(
        pl.kernel,
        mesh=mesh,
        compiler_params=pltpu.CompilerParams(
            use_tc_tiling_on_sc=False, needs_layout_passes=False),
        out_type=[
            jax.ShapeDtypeStruct((n_tokens * VP,), jnp.float32),
            jax.ShapeDtypeStruct((NW, LANES), jnp.float32),
        ],
        scratch_types=(
            [pltpu.VMEM((per_w + LANES,), jnp.int32)] * 2
            + [pltpu.VMEM((chunk, VP), jnp.float32)] * nbuf
            + [pltpu.VMEM((LANES,), jnp.float32)]
            + [pltpu.VMEM((LSE_PAD,), jnp.float32)]
            + [pltpu.SemaphoreType.DMA] * nbuf
            + [pltpu.SemaphoreType.DMA] * nbuf
        ),
    )
    def sc_gather(table_hbm, idx_hbm, tgt_hbm, lse_hbm, out_hbm, part_hbm,
                  idx_v, tgt_v, *rest):
        rows = rest[:nbuf]
        acc_v, lse_v = rest[nbuf], rest[nbuf + 1]
        gsem = rest[nbuf + 2:nbuf + 2 + nbuf]
        ssem = rest[nbuf + 2 + nbuf:]
        wid = lax.axis_index("s") * NUM_CORES + lax.axis_index("c")
        base_w = pl.multiple_of(wid * per_w, per_w)
        pltpu.sync_copy(lse_hbm, lse_v)
        pltpu.sync_copy(idx_hbm.at[pl.ds(base_w, per_w)],
                        idx_v.at[pl.ds(0, per_w)])
        pltpu.sync_copy(tgt_hbm.at[pl.ds(base_w, per_w)],
                        tgt_v.at[pl.ds(0, per_w)])
        acc_v[...] = jnp.zeros((LANES,), jnp.float32)

        def start_gather(g, b):
            off = pl.multiple_of(g * chunk, chunk)
            pltpu.async_copy(
                table_hbm.at[idx_v.at[pl.ds(off, chunk)]], rows[b], gsem[b])

        def wait_gather(b):
            pltpu.make_async_copy(
                table_hbm.at[pl.ds(0, chunk)], rows[b], gsem[b]).wait()

        def start_scatter(g, b):
            off = pl.multiple_of((base_w + g * chunk) * VP, VP)
            for i in range(chunk):
                pltpu.async_copy(
                    rows[b].at[i], out_hbm.at[pl.ds(off + i * VP, VP)], ssem[b])

        def wait_scatter(b):
            for i in range(chunk):
                pltpu.make_async_copy(
                    rows[b].at[i], out_hbm.at[pl.ds(0, VP)], ssem[b]).wait()

        def loss_partial(g, b):
            part = jnp.zeros((LANES,), jnp.float32)
            for j in range(groups):
                off = pl.multiple_of(g * chunk + j * LANES, 8)
                rid16 = lax.iota(jnp.int32, LANES) + (j * LANES)
                ok = rid16 < chunk
                rid16 = jnp.where(ok, rid16, 0)
                idx16 = jnp.where(ok, idx_v[pl.ds(off, LANES)], 0)
                tgt16 = jnp.where(ok, tgt_v[pl.ds(off, LANES)], 0)
                lse16 = plsc.load_gather(lse_v, [idx16])
                picked = plsc.load_gather(rows[b], [rid16, tgt16])
                part = part + jnp.where(ok, lse16 - picked, 0.0)
            acc_v[...] = acc_v[...] + part

        for b in range(nbuf):
            start_gather(b, b)

        def body(k, carry):
            for b in range(nbuf):
                g = k * nbuf + b
                wait_gather(b)
                start_scatter(g, b)
                loss_partial(g, b)

                @pl.when(g + nbuf < n_chunks)
                def _():
                    wait_scatter(b)
                    start_gather(g + nbuf, b)
            return carry

        lax.fori_loop(0, n_chunks // nbuf, body, 0)
        for b in range(nbuf):
            wait_scatter(b)
        pltpu.sync_copy(acc_v, part_hbm.at[wid])

    return sc_gather


def _detile_body(in_ref, out_ref, *, nb, tlen):
    x = in_ref[...]
    x3 = x.reshape(nb * tlen, VP // 128, 128)
    y = jnp.concatenate(
        [x3[:, c, :] for c in range(VP // 128)], axis=1)
    out_ref[...] = y[:, :V].reshape(nb, tlen, V)


def _detile(flat, b, t):
    nb = 8
    rows = nb * t * VP // 128
    x = flat.reshape(b * t * VP // 128, 128)
    return pl.pallas_call(
        functools.partial(_detile_body, nb=nb, tlen=t),
        grid=(b // nb,),
        in_specs=[pl.BlockSpec((rows, 128), lambda i: (i, 0))],
        out_specs=pl.BlockSpec((nb, t, V), lambda i: (i, 0, 0)),
        out_shape=jax.ShapeDtypeStruct((b, t, V), jnp.float32),
    )(x)


def kernel(index, targets, table):
    b, t = index.shape
    n_tokens = b * t
    idx_flat = index.reshape(n_tokens).astype(jnp.int32)
    tgt_flat = targets.reshape(n_tokens).astype(jnp.int32)
    table_pad = jnp.pad(table, ((0, 0), (0, VP - V)))
    lse = _compute_lse(table)
    sc_gather = _make_sc_gather(n_tokens, chunk=32)
    logits_flat, partials = sc_gather(table_pad, idx_flat, tgt_flat, lse)
    loss = _compute_loss(partials, n_tokens)
    return _detile(logits_flat, b, t), loss


# speedup vs baseline: 1.5380x; 1.1767x over previous
import functools

import jax
import jax.numpy as jnp
from jax import lax
from jax.experimental import pallas as pl
from jax.experimental.pallas import tpu as pltpu
from jax.experimental.pallas import tpu_sc as plsc

V = 1000
VP = 1024
LSE_PAD = 1024
NUM_CORES = 2
NUM_SUBCORES = 16
LANES = 16
NW = NUM_CORES * NUM_SUBCORES


def _lse_body(table_ref, out_ref):
    x = table_ref[...]
    m = jnp.max(x, axis=1, keepdims=True)
    s = jnp.sum(jnp.exp(x - m), axis=1, keepdims=True)
    lse = m + jnp.log(s)
    pad = jnp.zeros((LSE_PAD - V, 1), jnp.float32)
    out_ref[...] = jnp.concatenate([lse, pad], axis=0)


def _compute_lse(table):
    out = pl.pallas_call(
        _lse_body,
        out_shape=jax.ShapeDtypeStruct((LSE_PAD, 1), jnp.float32),
    )(table)
    return out.reshape(LSE_PAD)


def _loss_body(part_ref, out_ref, *, n_tokens):
    val = jnp.sum(part_ref[...]) * (1.0 / n_tokens)
    out_ref[...] = jnp.broadcast_to(val, (1, 1))


def _compute_loss(partials, n_tokens):
    out = pl.pallas_call(
        functools.partial(_loss_body, n_tokens=n_tokens),
        out_shape=jax.ShapeDtypeStruct((1, 1), jnp.float32),
    )(partials)
    return out[0, 0]


def _make_sc_gather(batch, tlen, chunk):
    n_tokens = batch * tlen
    assert n_tokens % (8 * NW) == 0
    per_w = n_tokens // NW
    nbuf = 2
    assert per_w % chunk == 0
    n_chunks = per_w // chunk
    assert n_chunks % nbuf == 0 and chunk % 8 == 0
    groups = (chunk + LANES - 1) // LANES

    mesh = plsc.VectorSubcoreMesh(
        core_axis_name="c", subcore_axis_name="s",
        num_cores=NUM_CORES, num_subcores=NUM_SUBCORES)

    @functools.partial(
        pl.kernel,
        mesh=mesh,
        compiler_params=pltpu.CompilerParams(
            use_tc_tiling_on_sc=False, needs_layout_passes=False),
        out_type=[
            jax.ShapeDtypeStruct((n_tokens * VP,), jnp.float32),
            jax.ShapeDtypeStruct((NW, LANES), jnp.float32),
        ],
        scratch_types=(
            [pltpu.VMEM((per_w + LANES,), jnp.int32)] * 2
            + [pltpu.VMEM((chunk, VP), jnp.float32)] * nbuf
            + [pltpu.VMEM((LANES,), jnp.float32)]
            + [pltpu.VMEM((LSE_PAD,), jnp.float32)]
            + [pltpu.SemaphoreType.DMA] * nbuf
            + [pltpu.SemaphoreType.DMA] * nbuf
        ),
    )
    def sc_gather(table_hbm, idx_hbm, tgt_hbm, lse_hbm, out_hbm, part_hbm,
                  idx_v, tgt_v, *rest):
        rows = rest[:nbuf]
        acc_v, lse_v = rest[nbuf], rest[nbuf + 1]
        gsem = rest[nbuf + 2:nbuf + 2 + nbuf]
        ssem = rest[nbuf + 2 + nbuf:]
        wid = lax.axis_index("s") * NUM_CORES + lax.axis_index("c")
        base_w = pl.multiple_of(wid * per_w, per_w)
        pltpu.sync_copy(lse_hbm, lse_v)
        pltpu.sync_copy(idx_hbm.at[pl.ds(base_w, per_w)],
                        idx_v.at[pl.ds(0, per_w)])
        pltpu.sync_copy(tgt_hbm.at[pl.ds(base_w, per_w)],
                        tgt_v.at[pl.ds(0, per_w)])
        acc_v[...] = jnp.zeros((LANES,), jnp.float32)

        def start_gather(g, b):
            off = pl.multiple_of(g * chunk, chunk)
            pltpu.async_copy(
                table_hbm.at[idx_v.at[pl.ds(off, chunk)]], rows[b], gsem[b])

        def wait_gather(b):
            pltpu.make_async_copy(
                table_hbm.at[pl.ds(0, chunk)], rows[b], gsem[b]).wait()

        def start_scatter(g, b):
            p0 = base_w + g * chunk
            for i in range(chunk):
                p = p0 + i
                off = ((p % tlen) * batch + p // tlen) * VP
                off = pl.multiple_of(off, VP)
                pltpu.async_copy(
                    rows[b].at[i], out_hbm.at[pl.ds(off, VP)], ssem[b])

        def wait_scatter(b):
            for i in range(chunk):
                pltpu.make_async_copy(
                    rows[b].at[i], out_hbm.at[pl.ds(0, VP)], ssem[b]).wait()

        def loss_partial(g, b):
            part = jnp.zeros((LANES,), jnp.float32)
            for j in range(groups):
                off = pl.multiple_of(g * chunk + j * LANES, 8)
                rid16 = lax.iota(jnp.int32, LANES) + (j * LANES)
                ok = rid16 < chunk
                rid16 = jnp.where(ok, rid16, 0)
                idx16 = jnp.where(ok, idx_v[pl.ds(off, LANES)], 0)
                tgt16 = jnp.where(ok, tgt_v[pl.ds(off, LANES)], 0)
                lse16 = plsc.load_gather(lse_v, [idx16])
                picked = plsc.load_gather(rows[b], [rid16, tgt16])
                part = part + jnp.where(ok, lse16 - picked, 0.0)
            acc_v[...] = acc_v[...] + part

        for b in range(nbuf):
            start_gather(b, b)

        def body(k, carry):
            for b in range(nbuf):
                g = k * nbuf + b
                wait_gather(b)
                start_scatter(g, b)
                loss_partial(g, b)

                @pl.when(g + nbuf < n_chunks)
                def _():
                    wait_scatter(b)
                    start_gather(g + nbuf, b)
            return carry

        lax.fori_loop(0, n_chunks // nbuf, body, 0)
        for b in range(nbuf):
            wait_scatter(b)
        pltpu.sync_copy(acc_v, part_hbm.at[wid])

    return sc_gather


def _detile_body(in_ref, out_ref):
    x3 = in_ref[...].reshape(128, VP // 128, 128)
    for c in range(V // 128):
        out_ref[0, pl.ds(c * 128, 128), :] = x3[:, c, :].T
    out_ref[0, pl.ds(896, 104), :] = x3[:, 7, :104].T


def _detile(flat, b, t):
    x = flat.reshape(b * t * VP // 128, 128)
    nbs = b // 128
    out = pl.pallas_call(
        _detile_body,
        grid=(t, nbs),
        in_specs=[pl.BlockSpec((1024, 128), lambda i, j: (i * nbs + j, 0))],
        out_specs=pl.BlockSpec((1, V, 128), lambda i, j: (i, 0, j)),
        out_shape=jax.ShapeDtypeStruct((t, V, b), jnp.float32),
    )(x)
    return out.transpose(2, 0, 1)


def kernel(index, targets, table):
    b, t = index.shape
    n_tokens = b * t
    idx_flat = index.reshape(n_tokens).astype(jnp.int32)
    tgt_flat = targets.reshape(n_tokens).astype(jnp.int32)
    table_pad = jnp.pad(table, ((0, 0), (0, VP - V)))
    lse = _compute_lse(table)
    sc_gather = _make_sc_gather(b, t, chunk=32)
    logits_flat, partials = sc_gather(table_pad, idx_flat, tgt_flat, lse)
    loss = _compute_loss(partials, n_tokens)
    return _detile(logits_flat, b, t), loss
